# baseline (speedup 1.0000x reference)
import functools

import jax
import jax.numpy as jnp
from jax import lax
from jax.experimental import pallas as pl
from jax.experimental.pallas import tpu as pltpu
from jax.experimental.pallas import tpu_sc as plsc

N = 10000
E = 320000
DH = 128
DE = 16
G = 256

NC = 2
NS = 16
NW = NC * NS
EPW = E // NW
CH = 80
NCH = EPW // CH
RCH = CH
NCHK = N // RCH
RROT = (NCHK + NS - 1) // NS

_mesh = plsc.VectorSubcoreMesh(core_axis_name="c", subcore_axis_name="s")


def _leaky(v):
    return jnp.where(v >= 0, v, 0.01 * v)


def _elu(v):
    return jnp.where(v > 0, v, jnp.exp(jnp.minimum(v, 0.0)) - 1.0)


def _tc_a_body(x_ref, w1t_ref, b1_ref, wg1nt_ref, wg2t_ref, attr_ref,
               x1_ref, u_ref, w_ref, rv_ref):
    x1 = _leaky(jnp.dot(x_ref[...], w1t_ref[...],
                        preferred_element_type=jnp.float32) + b1_ref[...])
    x1_ref[...] = x1
    u_ref[...] = jnp.dot(x1, wg1nt_ref[...], preferred_element_type=jnp.float32)
    w_ref[...] = jnp.dot(x1, wg2t_ref[...], preferred_element_type=jnp.float32)
    rv_ref[...] = jnp.sum(x1 * attr_ref[...], axis=1, keepdims=True)


def _tc_a(x, w1t, b1, wg1nt, wg2t, attr):
    rb = 1000
    grid = (N // rb,)
    return pl.pallas_call(
        _tc_a_body,
        grid=grid,
        in_specs=[
            pl.BlockSpec((rb, DH), lambda i: (i, 0)),
            pl.BlockSpec((DH, DH), lambda i: (0, 0)),
            pl.BlockSpec((DH,), lambda i: (0,)),
            pl.BlockSpec((DH, DH), lambda i: (0, 0)),
            pl.BlockSpec((DH, DH), lambda i: (0, 0)),
            pl.BlockSpec((DH,), lambda i: (0,)),
        ],
        out_specs=[
            pl.BlockSpec((rb, DH), lambda i: (i, 0)),
            pl.BlockSpec((rb, DH), lambda i: (i, 0)),
            pl.BlockSpec((rb, DH), lambda i: (i, 0)),
            pl.BlockSpec((rb, 1), lambda i: (i, 0)),
        ],
        out_shape=[
            jax.ShapeDtypeStruct((N, DH), jnp.float32),
            jax.ShapeDtypeStruct((N, DH), jnp.float32),
            jax.ShapeDtypeStruct((N, DH), jnp.float32),
            jax.ShapeDtypeStruct((N, 1), jnp.float32),
        ],
    )(x, w1t, b1, wg1nt, wg2t, attr)


def _tc_bv_body(ea_ref, wg1et_ref, v_ref):
    v_ref[...] = jnp.dot(ea_ref[...], wg1et_ref[...],
                         preferred_element_type=jnp.float32)


def _tc_bv(ea, wg1et):
    eb = 2560
    grid = (E // eb,)
    return pl.pallas_call(
        _tc_bv_body,
        grid=grid,
        in_specs=[
            pl.BlockSpec((eb, DE), lambda i: (i, 0)),
            pl.BlockSpec((DE, DH), lambda i: (0, 0)),
        ],
        out_specs=pl.BlockSpec((eb, DH), lambda i: (i, 0)),
        out_shape=jax.ShapeDtypeStruct((E, DH), jnp.float32),
    )(ea, wg1et)


def _gru_block(hin, hprev, wit, wht, bi, bh):
    gi = jnp.dot(hin, wit, preferred_element_type=jnp.float32) + bi
    gh = jnp.dot(hprev, wht, preferred_element_type=jnp.float32) + bh
    i_r, i_z, i_n = gi[:, :DH], gi[:, DH:2 * DH], gi[:, 2 * DH:]
    h_r, h_z, h_n = gh[:, :DH], gh[:, DH:2 * DH], gh[:, 2 * DH:]
    r = jax.nn.sigmoid(i_r + h_r)
    z = jax.nn.sigmoid(i_z + h_z)
    n = jnp.tanh(i_n + r * h_n)
    return (1.0 - z) * n + z * hprev


def _tc_c_body(hp0_ref, hp1_ref, s0_ref, s1_ref, gb_ref, x1_ref,
               wi0t_ref, wh0t_ref, bi0_ref, bh0_ref, gwt_ref, asrc_ref,
               adst_ref, x2_ref, xp_ref, av_ref, bv_ref):
    H = hp0_ref[...] + hp1_ref[...]
    s = s0_ref[...] + s1_ref[...]
    h = _elu(H / (s + 1e-16) + gb_ref[...])
    x2 = jax.nn.relu(_gru_block(h, x1_ref[...], wi0t_ref[...], wh0t_ref[...],
                                bi0_ref[...], bh0_ref[...]))
    x2_ref[...] = x2
    xp = jnp.dot(x2, gwt_ref[...], preferred_element_type=jnp.float32)
    xp_ref[...] = xp
    av_ref[...] = jnp.sum(xp * asrc_ref[...], axis=1, keepdims=True)
    bv_ref[...] = jnp.sum(xp * adst_ref[...], axis=1, keepdims=True)


def _tc_c(hp0, hp1, s0, s1, gb, x1, wi0t, wh0t, bi0, bh0, gwt, asrc, adst):
    rb = 1000
    grid = (N // rb,)
    return pl.pallas_call(
        _tc_c_body,
        grid=grid,
        in_specs=[
            pl.BlockSpec((rb, DH), lambda i: (i, 0)),
            pl.BlockSpec((rb, DH), lambda i: (i, 0)),
            pl.BlockSpec((rb, 1), lambda i: (i, 0)),
            pl.BlockSpec((rb, 1), lambda i: (i, 0)),
            pl.BlockSpec((DH,), lambda i: (0,)),
            pl.BlockSpec((rb, DH), lambda i: (i, 0)),
            pl.BlockSpec((DH, 3 * DH), lambda i: (0, 0)),
            pl.BlockSpec((DH, 3 * DH), lambda i: (0, 0)),
            pl.BlockSpec((3 * DH,), lambda i: (0,)),
            pl.BlockSpec((3 * DH,), lambda i: (0,)),
            pl.BlockSpec((DH, DH), lambda i: (0, 0)),
            pl.BlockSpec((DH,), lambda i: (0,)),
            pl.BlockSpec((DH,), lambda i: (0,)),
        ],
        out_specs=[
            pl.BlockSpec((rb, DH), lambda i: (i, 0)),
            pl.BlockSpec((rb, DH), lambda i: (i, 0)),
            pl.BlockSpec((rb, 1), lambda i: (i, 0)),
            pl.BlockSpec((rb, 1), lambda i: (i, 0)),
        ],
        out_shape=[
            jax.ShapeDtypeStruct((N, DH), jnp.float32),
            jax.ShapeDtypeStruct((N, DH), jnp.float32),
            jax.ShapeDtypeStruct((N, 1), jnp.float32),
            jax.ShapeDtypeStruct((N, 1), jnp.float32),
        ],
    )(hp0, hp1, s0, s1, gb, x1, wi0t, wh0t, bi0, bh0, gwt, asrc, adst)


def _tc_d_body(hp0_ref, hp1_ref, s0_ref, s1_ref, gb_ref, x2_ref,
               wi1t_ref, wh1t_ref, bi1_ref, bh1_ref, w2t_ref, b2_ref,
               bf_ref, out_ref):
    H = hp0_ref[...] + hp1_ref[...]
    s = s0_ref[...] + s1_ref[...]
    h2 = _elu(H / (s + 1e-16) + gb_ref[...])
    x3 = jax.nn.relu(_gru_block(h2, x2_ref[...], wi1t_ref[...], wh1t_ref[...],
                                bi1_ref[...], bh1_ref[...]))
    node = jnp.dot(x3, w2t_ref[...], preferred_element_type=jnp.float32) + b2_ref[...]
    gids = lax.broadcasted_iota(jnp.int32, (1, G), 1).astype(jnp.float32)
    oh = (bf_ref[...] == gids).astype(jnp.float32)
    contrib = lax.dot_general(oh, node, (((0,), (0,)), ((), ())),
                              preferred_element_type=jnp.float32)

    @pl.when(pl.program_id(0) == 0)
    def _init():
        out_ref[...] = jnp.zeros_like(out_ref)

    out_ref[...] += contrib


def _tc_d(hp0, hp1, s0, s1, gb, x2, wi1t, wh1t, bi1, bh1, w2t, b2, batch_f):
    rb = 1000
    grid = (N // rb,)
    return pl.pallas_call(
        _tc_d_body,
        grid=grid,
        in_specs=[
            pl.BlockSpec((rb, DH), lambda i: (i, 0)),
            pl.BlockSpec((rb, DH), lambda i: (i, 0)),
            pl.BlockSpec((rb, 1), lambda i: (i, 0)),
            pl.BlockSpec((rb, 1), lambda i: (i, 0)),
            pl.BlockSpec((DH,), lambda i: (0,)),
            pl.BlockSpec((rb, DH), lambda i: (i, 0)),
            pl.BlockSpec((DH, 3 * DH), lambda i: (0, 0)),
            pl.BlockSpec((DH, 3 * DH), lambda i: (0, 0)),
            pl.BlockSpec((3 * DH,), lambda i: (0,)),
            pl.BlockSpec((3 * DH,), lambda i: (0,)),
            pl.BlockSpec((DH, DH), lambda i: (0, 0)),
            pl.BlockSpec((DH,), lambda i: (0,)),
            pl.BlockSpec((rb, 1), lambda i: (i, 0)),
        ],
        out_specs=pl.BlockSpec((G, DH), lambda i: (0, 0)),
        out_shape=jax.ShapeDtypeStruct((G, DH), jnp.float32),
    )(hp0, hp1, s0, s1, gb, x2, wi1t, wh1t, bi1, bh1, w2t, b2, batch_f)


CH1 = 40
NCH1 = EPW // CH1


def _sc_fused1_body(u_hbm, w_hbm, v_hbm, ei_hbm, rd_hbm, attl_hbm,
                    z2_hbm, z1_hbm, H_out, s_out,
                    sds0, sds1, sdd0, sdd1, ru0, ru1, rv0, rv1, rw0, rw1,
                    cf0, cf1, attl_v, tb_v, H_sh, s_sh,
                    smg0, smg1, smv0, smv1, smw0, smw1, smh0, smh1,
                    sms0, sms1):
    cid = lax.axis_index("c")
    sid = lax.axis_index("s")
    wid = sid * NC + cid
    base = wid * EPW
    crow0 = wid * NCH1

    @pl.when(sid == 0)
    def _zs():
        pltpu.sync_copy(z1_hbm, s_sh)

    @pl.when(sid < NZC)
    def _zh():
        pltpu.sync_copy(z2_hbm, H_sh.at[pl.ds(sid * ZR, ZR)])

    pltpu.sync_copy(rd_hbm, tb_v)
    pltpu.sync_copy(attl_hbm, attl_v)
    al = [attl_v[pl.ds(j * 16, 16)] for j in range(DH // 16)]
    lane = lax.iota(jnp.int32, 16)
    m0 = lane == 0

    plsc.subcore_barrier()

    banks = ((sds0, sdd0, ru0, rv0, rw0, cf0, smg0, smv0, smw0, smh0, sms0),
             (sds1, sdd1, ru1, rv1, rw1, cf1, smg1, smv1, smw1, smh1, sms1))

    def drain_scatters(bk):
        _, sdd, _, _, rw, cf, _, _, _, smh, sms = bk
        pltpu.make_async_copy(cf, s_sh.at[sdd], sms).wait()
        pltpu.make_async_copy(rw, H_sh.at[sdd], smh).wait()

    def fire(c, bk, dr):
        sds, sdd, ru, rv, rw, _, smg, smv, smw, _, _ = bk
        if dr:
            drain_scatters(bk)
        pltpu.sync_copy(ei_hbm.at[0, crow0 + c], sds)
        pltpu.sync_copy(ei_hbm.at[1, crow0 + c], sdd)
        pltpu.async_copy(u_hbm.at[sds], ru, smg)
        pltpu.async_copy(v_hbm.at[pl.ds(base + c * CH1, CH1)], rv, smv)
        pltpu.async_copy(w_hbm.at[sds], rw, smw)

    def process(c, bk):
        sds, sdd, ru, rv, rw, cf, smg, smv, smw, smh, sms = bk
        pltpu.make_async_copy(u_hbm.at[sds], ru, smg).wait()
        pltpu.make_async_copy(v_hbm.at[pl.ds(base + c * CH1, CH1)], rv,
                              smv).wait()

        @plsc.parallel_loop(0, CH1, unroll=4)
        def _dot(i):
            acc = _leaky(ru[i, pl.ds(0, 16)] + rv[i, pl.ds(0, 16)]) * al[0]
            for j in range(1, DH // 16):
                acc = acc + _leaky(ru[i, pl.ds(j * 16, 16)]
                                   + rv[i, pl.ds(j * 16, 16)]) * al[j]
            t = jnp.sum(acc)
            isp = jnp.full((16,), i, jnp.int32)
            dsp = plsc.load_gather(sdd, [isp])
            rd = plsc.load_gather(tb_v, [dsp])
            a = t + rd
            ev = jnp.exp(jnp.where(a >= 0, a, 0.01 * a))
            plsc.store_scatter(cf, [isp], ev, mask=m0)

        pltpu.async_copy(cf, s_sh.at[sdd], sms, add=True)
        pltpu.make_async_copy(w_hbm.at[sds], rw, smw).wait()

        @plsc.parallel_loop(0, CH1, unroll=4)
        def _scale(i):
            bc = plsc.load_gather(cf, [jnp.full((16,), i, jnp.int32)])
            for j in range(DH // 16):
                rw[i, pl.ds(j * 16, 16)] = rw[i, pl.ds(j * 16, 16)] * bc

        pltpu.async_copy(rw, H_sh.at[sdd], smh, add=True)

    fire(0, banks[0], False)
    fire(1, banks[1], False)

    def pair(i, carry):
        c0 = 2 * i
        process(c0, banks[0])

        @pl.when(c0 + 2 < NCH1)
        def _n0():
            fire(c0 + 2, banks[0], True)

        process(c0 + 1, banks[1])

        @pl.when(c0 + 3 < NCH1)
        def _n1():
            fire(c0 + 3, banks[1], True)

        return carry

    lax.fori_loop(0, NCH1 // 2, pair, 0)
    drain_scatters(banks[0])
    drain_scatters(banks[1])

    plsc.subcore_barrier()

    @pl.when(sid == 0)
    def _souts():
        pltpu.sync_copy(s_sh, s_out.at[cid])

    @pl.when(sid < NZC)
    def _hout():
        pltpu.sync_copy(H_sh.at[pl.ds(sid * ZR, ZR)],
                        H_out.at[cid, pl.ds(sid * ZR, ZR)])


_sc_fused1 = pl.kernel(
    _sc_fused1_body,
    out_type=(
        jax.ShapeDtypeStruct((NC, N, DH), jnp.float32),
        jax.ShapeDtypeStruct((NC, N), jnp.float32),
    ),
    mesh=_mesh,
    compiler_params=pltpu.CompilerParams(needs_layout_passes=False),
    scratch_types=(
        [pltpu.VMEM((CH1,), jnp.int32)] * 4
        + [pltpu.VMEM((CH1, DH), jnp.float32)] * 6
        + [pltpu.VMEM((CH1,), jnp.float32)] * 2
        + [
            pltpu.VMEM((DH,), jnp.float32),
            pltpu.VMEM((N,), jnp.float32),
            pltpu.VMEM_SHARED((N, DH), jnp.float32),
            pltpu.VMEM_SHARED((N,), jnp.float32),
        ]
        + [pltpu.SemaphoreType.DMA] * 10
    ),
)


ZR = 1000
NZC = N // ZR


def _sc_edge_body(ta_hbm, tb_hbm, ei_hbm, w_hbm, z2_hbm, z1_hbm,
                  H_out, s_out, sds0, sds1, sdd0, sdd1, coef0, coef1,
                  rows0, rows1, ta_v, tb_v, H_sh, s_sh, sem0, sem1,
                  semh0, semh1, sems0, sems1):
    cid = lax.axis_index("c")
    sid = lax.axis_index("s")
    wid = sid * NC + cid
    crow0 = wid * NCH

    @pl.when(sid == 0)
    def _zs():
        pltpu.sync_copy(z1_hbm, s_sh)

    @pl.when(sid < NZC)
    def _zh():
        pltpu.sync_copy(z2_hbm, H_sh.at[pl.ds(sid * ZR, ZR)])

    pltpu.sync_copy(ta_hbm, ta_v)
    pltpu.sync_copy(tb_hbm, tb_v)

    plsc.subcore_barrier()

    banks = ((sds0, sdd0, coef0, rows0, sem0, semh0, sems0),
             (sds1, sdd1, coef1, rows1, sem1, semh1, sems1))

    def drain_scatters(bk):
        _, sdd, coef, rr, _, smh, sms = bk
        pltpu.make_async_copy(coef, s_sh.at[sdd], sms).wait()
        pltpu.make_async_copy(rr, H_sh.at[sdd], smh).wait()

    def fire(c, bk, dr):
        sds, sdd, _, rr, smg, _, _ = bk
        if dr:
            drain_scatters(bk)
        pltpu.sync_copy(ei_hbm.at[0, crow0 + c], sds)
        pltpu.sync_copy(ei_hbm.at[1, crow0 + c], sdd)
        pltpu.async_copy(w_hbm.at[sds], rr, smg)

    def process(bk):
        sds, sdd, coef, rr, smg, smh, sms = bk
        for g in range(CH // 16):
            dv = sdd[pl.ds(g * 16, 16)]
            sv = sds[pl.ds(g * 16, 16)]
            a = plsc.load_gather(ta_v, [sv]) + plsc.load_gather(tb_v, [dv])
            ev = jnp.exp(jnp.where(a >= 0, a, 0.01 * a))
            coef[pl.ds(g * 16, 16)] = ev
        pltpu.async_copy(coef, s_sh.at[sdd], sms, add=True)
        pltpu.make_async_copy(w_hbm.at[sds], rr, smg).wait()

        @plsc.parallel_loop(0, CH, unroll=8)
        def _scale(i):
            bc = plsc.load_gather(coef, [jnp.full((16,), 0, jnp.int32) + i])
            for j in range(DH // 16):
                rr[i, pl.ds(j * 16, 16)] = rr[i, pl.ds(j * 16, 16)] * bc

        pltpu.async_copy(rr, H_sh.at[sdd], smh, add=True)

    fire(0, banks[0], False)
    fire(1, banks[1], False)

    def pair(i, carry):
        c0 = 2 * i
        process(banks[0])

        @pl.when(c0 + 2 < NCH)
        def _n0():
            fire(c0 + 2, banks[0], True)

        process(banks[1])

        @pl.when(c0 + 3 < NCH)
        def _n1():
            fire(c0 + 3, banks[1], True)

        return carry

    lax.fori_loop(0, NCH // 2, pair, 0)
    if NCH % 2:
        process(banks[0])
    drain_scatters(banks[0])
    drain_scatters(banks[1])

    plsc.subcore_barrier()

    @pl.when(sid == 0)
    def _souts():
        pltpu.sync_copy(s_sh, s_out.at[cid])

    @pl.when(sid < NZC)
    def _hout():
        pltpu.sync_copy(H_sh.at[pl.ds(sid * ZR, ZR)],
                        H_out.at[cid, pl.ds(sid * ZR, ZR)])


_sc_edge2 = pl.kernel(
    _sc_edge_body,
    out_type=(
        jax.ShapeDtypeStruct((NC, N, DH), jnp.float32),
        jax.ShapeDtypeStruct((NC, N), jnp.float32),
    ),
    mesh=_mesh,
    compiler_params=pltpu.CompilerParams(needs_layout_passes=False),
    scratch_types=[
        pltpu.VMEM((CH,), jnp.int32),
        pltpu.VMEM((CH,), jnp.int32),
        pltpu.VMEM((CH,), jnp.int32),
        pltpu.VMEM((CH,), jnp.int32),
        pltpu.VMEM((CH,), jnp.float32),
        pltpu.VMEM((CH,), jnp.float32),
        pltpu.VMEM((CH, DH), jnp.float32),
        pltpu.VMEM((CH, DH), jnp.float32),
        pltpu.VMEM((N,), jnp.float32),
        pltpu.VMEM((N,), jnp.float32),
        pltpu.VMEM_SHARED((N, DH), jnp.float32),
        pltpu.VMEM_SHARED((N,), jnp.float32),
    ] + [pltpu.SemaphoreType.DMA] * 6,
)


def kernel(x, edge_index, edge_attr, batch, W_lin1, b_lin1, Wg1, Wg2, att_l,
           att_r, gate_bias, Wi0, Wh0, bi0, bh0, gat_W, att_src, att_dst,
           gat_bias, Wi1, Wh1, bi1, bh1, W_lin2, b_lin2):
    src = edge_index[0]
    dst = edge_index[1]
    batch_f = batch.astype(jnp.float32).reshape(N, 1)
    zH = jnp.zeros((ZR, DH), jnp.float32)
    z1 = jnp.zeros((N,), jnp.float32)
    ei1 = edge_index.reshape(2, E // CH1, CH1)
    ei2 = edge_index.reshape(2, E // CH, CH)

    x1, u, w, rv = _tc_a(x, W_lin1.T, b_lin1, Wg1[:, :DH].T, Wg2.T, att_r)
    v = _tc_bv(edge_attr, Wg1[:, DH:].T)
    Hp, sp = _sc_fused1(u, w, v, ei1, rv.reshape(N), att_l, zH, z1)
    x2, xp, av, bv = _tc_c(Hp[0], Hp[1], sp[0].reshape(N, 1),
                           sp[1].reshape(N, 1), gate_bias, x1, Wi0.T, Wh0.T,
                           bi0, bh0, gat_W.T, att_src, att_dst)
    H2p, s2p = _sc_edge2(av.reshape(N), bv.reshape(N), ei2, xp, zH, z1)
    out = _tc_d(H2p[0], H2p[1], s2p[0].reshape(N, 1), s2p[1].reshape(N, 1),
                gat_bias, x2, Wi1.T, Wh1.T, bi1, bh1, W_lin2.T, b_lin2,
                batch_f)
    return out


# speedup vs baseline: 1.2282x; 1.2282x over previous
import functools

import jax
import jax.numpy as jnp
from jax import lax
from jax.experimental import pallas as pl
from jax.experimental.pallas import tpu as pltpu
from jax.experimental.pallas import tpu_sc as plsc

N = 10000
E = 320000
DH = 128
DE = 16
G = 256

NC = 2
NS = 16
NW = NC * NS
EPW = E // NW
CH = 80
NCH = EPW // CH
RCH = CH
NCHK = N // RCH
RROT = (NCHK + NS - 1) // NS

_mesh = plsc.VectorSubcoreMesh(core_axis_name="c", subcore_axis_name="s")


def _leaky(v):
    return jnp.where(v >= 0, v, 0.01 * v)


def _elu(v):
    return jnp.where(v > 0, v, jnp.exp(jnp.minimum(v, 0.0)) - 1.0)


def _tc_a_body(x_ref, w1t_ref, b1_ref, wg1nt_ref, wg2t_ref, attr_ref,
               x1_ref, u_ref, w_ref, rv_ref):
    x1 = _leaky(jnp.dot(x_ref[...], w1t_ref[...],
                        preferred_element_type=jnp.float32) + b1_ref[...])
    x1_ref[...] = x1
    u_ref[...] = jnp.dot(x1, wg1nt_ref[...], preferred_element_type=jnp.float32)
    w_ref[...] = jnp.dot(x1, wg2t_ref[...], preferred_element_type=jnp.float32)
    rv_ref[...] = jnp.sum(x1 * attr_ref[...], axis=1, keepdims=True)


def _tc_a(x, w1t, b1, wg1nt, wg2t, attr):
    rb = 1000
    grid = (N // rb,)
    return pl.pallas_call(
        _tc_a_body,
        grid=grid,
        in_specs=[
            pl.BlockSpec((rb, DH), lambda i: (i, 0)),
            pl.BlockSpec((DH, DH), lambda i: (0, 0)),
            pl.BlockSpec((DH,), lambda i: (0,)),
            pl.BlockSpec((DH, DH), lambda i: (0, 0)),
            pl.BlockSpec((DH, DH), lambda i: (0, 0)),
            pl.BlockSpec((DH,), lambda i: (0,)),
        ],
        out_specs=[
            pl.BlockSpec((rb, DH), lambda i: (i, 0)),
            pl.BlockSpec((rb, DH), lambda i: (i, 0)),
            pl.BlockSpec((rb, DH), lambda i: (i, 0)),
            pl.BlockSpec((rb, 1), lambda i: (i, 0)),
        ],
        out_shape=[
            jax.ShapeDtypeStruct((N, DH), jnp.float32),
            jax.ShapeDtypeStruct((N, DH), jnp.float32),
            jax.ShapeDtypeStruct((N, DH), jnp.float32),
            jax.ShapeDtypeStruct((N, 1), jnp.float32),
        ],
    )(x, w1t, b1, wg1nt, wg2t, attr)


def _tc_b_body(gu_ref, eat_ref, wg1et_ref, attl_ref, t_ref):
    v = lax.dot_general(eat_ref[...], wg1et_ref[...],
                        (((0,), (0,)), ((), ())),
                        preferred_element_type=jnp.float32)
    hj = _leaky(gu_ref[...] + v)
    t_ref[...] = jnp.sum(hj * attl_ref[...], axis=1, keepdims=True)


def _tc_b(gu, eat, wg1et, attl):
    eb = 2560
    grid = (E // eb,)
    return pl.pallas_call(
        _tc_b_body,
        grid=grid,
        in_specs=[
            pl.BlockSpec((eb, DH), lambda i: (i, 0)),
            pl.BlockSpec((DE, eb), lambda i: (0, i)),
            pl.BlockSpec((DE, DH), lambda i: (0, 0)),
            pl.BlockSpec((DH,), lambda i: (0,)),
        ],
        out_specs=pl.BlockSpec((eb, 1), lambda i: (i, 0)),
        out_shape=jax.ShapeDtypeStruct((E, 1), jnp.float32),
    )(gu, eat, wg1et, attl)


def _gru_block(hin, hprev, wit, wht, bi, bh):
    gi = jnp.dot(hin, wit, preferred_element_type=jnp.float32) + bi
    gh = jnp.dot(hprev, wht, preferred_element_type=jnp.float32) + bh
    i_r, i_z, i_n = gi[:, :DH], gi[:, DH:2 * DH], gi[:, 2 * DH:]
    h_r, h_z, h_n = gh[:, :DH], gh[:, DH:2 * DH], gh[:, 2 * DH:]
    r = jax.nn.sigmoid(i_r + h_r)
    z = jax.nn.sigmoid(i_z + h_z)
    n = jnp.tanh(i_n + r * h_n)
    return (1.0 - z) * n + z * hprev


def _tc_c_body(hp0_ref, hp1_ref, s0_ref, s1_ref, gb_ref, x1_ref,
               wi0t_ref, wh0t_ref, bi0_ref, bh0_ref, gwt_ref, asrc_ref,
               adst_ref, x2_ref, xp_ref, av_ref, bv_ref):
    H = hp0_ref[...] + hp1_ref[...]
    s = s0_ref[...] + s1_ref[...]
    h = _elu(H / (s + 1e-16) + gb_ref[...])
    x2 = jax.nn.relu(_gru_block(h, x1_ref[...], wi0t_ref[...], wh0t_ref[...],
                                bi0_ref[...], bh0_ref[...]))
    x2_ref[...] = x2
    xp = jnp.dot(x2, gwt_ref[...], preferred_element_type=jnp.float32)
    xp_ref[...] = xp
    av_ref[...] = jnp.sum(xp * asrc_ref[...], axis=1, keepdims=True)
    bv_ref[...] = jnp.sum(xp * adst_ref[...], axis=1, keepdims=True)


def _tc_c(hp0, hp1, s0, s1, gb, x1, wi0t, wh0t, bi0, bh0, gwt, asrc, adst):
    rb = 1000
    grid = (N // rb,)
    return pl.pallas_call(
        _tc_c_body,
        grid=grid,
        in_specs=[
            pl.BlockSpec((rb, DH), lambda i: (i, 0)),
            pl.BlockSpec((rb, DH), lambda i: (i, 0)),
            pl.BlockSpec((rb, 1), lambda i: (i, 0)),
            pl.BlockSpec((rb, 1), lambda i: (i, 0)),
            pl.BlockSpec((DH,), lambda i: (0,)),
            pl.BlockSpec((rb, DH), lambda i: (i, 0)),
            pl.BlockSpec((DH, 3 * DH), lambda i: (0, 0)),
            pl.BlockSpec((DH, 3 * DH), lambda i: (0, 0)),
            pl.BlockSpec((3 * DH,), lambda i: (0,)),
            pl.BlockSpec((3 * DH,), lambda i: (0,)),
            pl.BlockSpec((DH, DH), lambda i: (0, 0)),
            pl.BlockSpec((DH,), lambda i: (0,)),
            pl.BlockSpec((DH,), lambda i: (0,)),
        ],
        out_specs=[
            pl.BlockSpec((rb, DH), lambda i: (i, 0)),
            pl.BlockSpec((rb, DH), lambda i: (i, 0)),
            pl.BlockSpec((rb, 1), lambda i: (i, 0)),
            pl.BlockSpec((rb, 1), lambda i: (i, 0)),
        ],
        out_shape=[
            jax.ShapeDtypeStruct((N, DH), jnp.float32),
            jax.ShapeDtypeStruct((N, DH), jnp.float32),
            jax.ShapeDtypeStruct((N, 1), jnp.float32),
            jax.ShapeDtypeStruct((N, 1), jnp.float32),
        ],
    )(hp0, hp1, s0, s1, gb, x1, wi0t, wh0t, bi0, bh0, gwt, asrc, adst)


def _tc_d_body(hp0_ref, hp1_ref, s0_ref, s1_ref, gb_ref, x2_ref,
               wi1t_ref, wh1t_ref, bi1_ref, bh1_ref, w2t_ref, b2_ref,
               bf_ref, out_ref):
    H = hp0_ref[...] + hp1_ref[...]
    s = s0_ref[...] + s1_ref[...]
    h2 = _elu(H / (s + 1e-16) + gb_ref[...])
    x3 = jax.nn.relu(_gru_block(h2, x2_ref[...], wi1t_ref[...], wh1t_ref[...],
                                bi1_ref[...], bh1_ref[...]))
    node = jnp.dot(x3, w2t_ref[...], preferred_element_type=jnp.float32) + b2_ref[...]
    gids = lax.broadcasted_iota(jnp.int32, (1, G), 1).astype(jnp.float32)
    oh = (bf_ref[...] == gids).astype(jnp.float32)
    contrib = lax.dot_general(oh, node, (((0,), (0,)), ((), ())),
                              preferred_element_type=jnp.float32)

    @pl.when(pl.program_id(0) == 0)
    def _init():
        out_ref[...] = jnp.zeros_like(out_ref)

    out_ref[...] += contrib


def _tc_d(hp0, hp1, s0, s1, gb, x2, wi1t, wh1t, bi1, bh1, w2t, b2, batch_f):
    rb = 1000
    grid = (N // rb,)
    return pl.pallas_call(
        _tc_d_body,
        grid=grid,
        in_specs=[
            pl.BlockSpec((rb, DH), lambda i: (i, 0)),
            pl.BlockSpec((rb, DH), lambda i: (i, 0)),
            pl.BlockSpec((rb, 1), lambda i: (i, 0)),
            pl.BlockSpec((rb, 1), lambda i: (i, 0)),
            pl.BlockSpec((DH,), lambda i: (0,)),
            pl.BlockSpec((rb, DH), lambda i: (i, 0)),
            pl.BlockSpec((DH, 3 * DH), lambda i: (0, 0)),
            pl.BlockSpec((DH, 3 * DH), lambda i: (0, 0)),
            pl.BlockSpec((3 * DH,), lambda i: (0,)),
            pl.BlockSpec((3 * DH,), lambda i: (0,)),
            pl.BlockSpec((DH, DH), lambda i: (0, 0)),
            pl.BlockSpec((DH,), lambda i: (0,)),
            pl.BlockSpec((rb, 1), lambda i: (i, 0)),
        ],
        out_specs=pl.BlockSpec((G, DH), lambda i: (0, 0)),
        out_shape=jax.ShapeDtypeStruct((G, DH), jnp.float32),
    )(hp0, hp1, s0, s1, gb, x2, wi1t, wh1t, bi1, bh1, w2t, b2, batch_f)


_GB = 4


def _sc_gather_body(table_hbm, idx_hbm, out_hbm,
                    ix0, ix1, ix2, ix3, r0, r1, r2, r3, s0, s1, s2, s3):
    wid = lax.axis_index("s") * NC + lax.axis_index("c")
    base = wid * EPW
    banks = ((ix0, r0, s0), (ix1, r1, s1), (ix2, r2, s2), (ix3, r3, s3))

    def fire(c, bk):
        ix, rr, sm = bk
        pltpu.sync_copy(idx_hbm.at[pl.ds(base + c * CH, CH)], ix)
        pltpu.async_copy(table_hbm.at[ix], rr, sm)

    def drain(c, bk):
        ix, rr, sm = bk
        pltpu.make_async_copy(table_hbm.at[ix], rr, sm).wait()
        pltpu.sync_copy(rr, out_hbm.at[pl.ds(base + c * CH, CH)])

    for b in range(_GB):
        fire(b, banks[b])

    def grp(i, carry):
        c0 = _GB * i
        for b in range(_GB):
            drain(c0 + b, banks[b])

            @pl.when(c0 + b + _GB < NCH)
            def _refire(b=b, c0=c0):
                fire(c0 + b + _GB, banks[b])
        return carry

    lax.fori_loop(0, NCH // _GB, grp, 0)
    drain(NCH - 1, banks[(NCH - 1) % _GB])


_sc_gather = pl.kernel(
    _sc_gather_body,
    out_type=jax.ShapeDtypeStruct((E, DH), jnp.float32),
    mesh=_mesh,
    compiler_params=pltpu.CompilerParams(needs_layout_passes=False),
    scratch_types=(
        [pltpu.VMEM((CH,), jnp.int32)] * _GB
        + [pltpu.VMEM((CH, DH), jnp.float32)] * _GB
        + [pltpu.SemaphoreType.DMA] * _GB
    ),
)


ZR = 1000
NZC = N // ZR


def _sc_edge_body(mode, ta_hbm, tb_hbm, cmb_hbm, w_hbm, z2_hbm, z1_hbm,
                  H_out, s_out, cmb0, cmb1, coef0, coef1,
                  rows0, rows1, ta_v, tb_v, H_sh, s_sh, sem0, sem1,
                  semh0, semh1, sems0, sems1):
    cid = lax.axis_index("c")
    sid = lax.axis_index("s")
    wid = sid * NC + cid
    crow0 = wid * NCH

    @pl.when(sid == 0)
    def _zs():
        pltpu.sync_copy(z1_hbm, s_sh)

    @pl.when(sid < NZC)
    def _zh():
        pltpu.sync_copy(z2_hbm, H_sh.at[pl.ds(sid * ZR, ZR)])

    if mode == 1:
        pltpu.sync_copy(ta_hbm, ta_v)
    pltpu.sync_copy(tb_hbm, tb_v)

    plsc.subcore_barrier()

    banks = ((cmb0, coef0, rows0, sem0, semh0, sems0),
             (cmb1, coef1, rows1, sem1, semh1, sems1))

    def drain_scatters(bk):
        cmb, coef, rr, _, smh, sms = bk
        pltpu.make_async_copy(coef, s_sh.at[cmb.at[1]], sms).wait()
        pltpu.make_async_copy(rr, H_sh.at[cmb.at[1]], smh).wait()

    def fire(c, bk, dr):
        cmb, _, rr, smg, _, _ = bk
        if dr:
            drain_scatters(bk)
        pltpu.sync_copy(cmb_hbm.at[crow0 + c], cmb)
        pltpu.async_copy(w_hbm.at[cmb.at[0]], rr, smg)

    def process(bk):
        cmb, coef, rr, smg, smh, sms = bk
        for g in range(CH // 16):
            dv = cmb[1, pl.ds(g * 16, 16)]
            if mode == 0:
                tv = plsc.bitcast(cmb[2, pl.ds(g * 16, 16)], jnp.float32)
                a = tv + plsc.load_gather(tb_v, [dv])
            else:
                sv = cmb[0, pl.ds(g * 16, 16)]
                a = plsc.load_gather(ta_v, [sv]) + plsc.load_gather(tb_v, [dv])
            ev = jnp.exp(jnp.where(a >= 0, a, 0.01 * a))
            coef[pl.ds(g * 16, 16)] = ev
        pltpu.async_copy(coef, s_sh.at[cmb.at[1]], sms, add=True)
        pltpu.make_async_copy(w_hbm.at[cmb.at[0]], rr, smg).wait()

        @plsc.parallel_loop(0, CH, unroll=8)
        def _scale(i):
            bc = plsc.load_gather(coef, [jnp.full((16,), 0, jnp.int32) + i])
            for j in range(DH // 16):
                rr[i, pl.ds(j * 16, 16)] = rr[i, pl.ds(j * 16, 16)] * bc

        pltpu.async_copy(rr, H_sh.at[cmb.at[1]], smh, add=True)

    fire(0, banks[0], False)
    fire(1, banks[1], False)

    def pair(i, carry):
        c0 = 2 * i
        process(banks[0])

        @pl.when(c0 + 2 < NCH)
        def _n0():
            fire(c0 + 2, banks[0], True)

        process(banks[1])

        @pl.when(c0 + 3 < NCH)
        def _n1():
            fire(c0 + 3, banks[1], True)

        return carry

    lax.fori_loop(0, NCH // 2, pair, 0)
    if NCH % 2:
        process(banks[0])
    drain_scatters(banks[0])
    drain_scatters(banks[1])

    plsc.subcore_barrier()

    @pl.when(sid == 0)
    def _souts():
        pltpu.sync_copy(s_sh, s_out.at[cid])

    @pl.when(sid < NZC)
    def _hout():
        pltpu.sync_copy(H_sh.at[pl.ds(sid * ZR, ZR)],
                        H_out.at[cid, pl.ds(sid * ZR, ZR)])


def _make_sc_edge(mode):
    ta_shape = (pltpu.VMEM((8,), jnp.float32) if mode == 0
                else pltpu.VMEM((N,), jnp.float32))
    rows = 3 if mode == 0 else 2
    return pl.kernel(
        functools.partial(_sc_edge_body, mode),
        out_type=(
            jax.ShapeDtypeStruct((NC, N, DH), jnp.float32),
            jax.ShapeDtypeStruct((NC, N), jnp.float32),
        ),
        mesh=_mesh,
        compiler_params=pltpu.CompilerParams(needs_layout_passes=False),
        scratch_types=[
            pltpu.VMEM((rows, CH), jnp.int32),
            pltpu.VMEM((rows, CH), jnp.int32),
            pltpu.VMEM((CH,), jnp.float32),
            pltpu.VMEM((CH,), jnp.float32),
            pltpu.VMEM((CH, DH), jnp.float32),
            pltpu.VMEM((CH, DH), jnp.float32),
            ta_shape,
            pltpu.VMEM((N,), jnp.float32),
            pltpu.VMEM_SHARED((N, DH), jnp.float32),
            pltpu.VMEM_SHARED((N,), jnp.float32),
        ] + [pltpu.SemaphoreType.DMA] * 6,
    )


_sc_edge1 = _make_sc_edge(0)
_sc_edge2 = _make_sc_edge(1)


def kernel(x, edge_index, edge_attr, batch, W_lin1, b_lin1, Wg1, Wg2, att_l,
           att_r, gate_bias, Wi0, Wh0, bi0, bh0, gat_W, att_src, att_dst,
           gat_bias, Wi1, Wh1, bi1, bh1, W_lin2, b_lin2):
    src = edge_index[0]
    dst = edge_index[1]
    batch_f = batch.astype(jnp.float32).reshape(N, 1)
    zH = jnp.zeros((ZR, DH), jnp.float32)
    z1 = jnp.zeros((N,), jnp.float32)
    dummy = jnp.zeros((8,), jnp.float32)
    src2 = src.reshape(-1, CH)
    dst2 = dst.reshape(-1, CH)
    cmb2 = jnp.stack([src2, dst2], axis=1)

    x1, u, w, rv = _tc_a(x, W_lin1.T, b_lin1, Wg1[:, :DH].T, Wg2.T, att_r)
    gu = _sc_gather(u, src)
    t = _tc_b(gu, edge_attr.T, Wg1[:, DH:].T, att_l)
    tbits = lax.bitcast_convert_type(t.reshape(-1, CH), jnp.int32)
    cmb1 = jnp.stack([src2, dst2, tbits], axis=1)
    Hp, sp = _sc_edge1(dummy, rv.reshape(N), cmb1, w, zH, z1)
    x2, xp, av, bv = _tc_c(Hp[0], Hp[1], sp[0].reshape(N, 1),
                           sp[1].reshape(N, 1), gate_bias, x1, Wi0.T, Wh0.T,
                           bi0, bh0, gat_W.T, att_src, att_dst)
    H2p, s2p = _sc_edge2(av.reshape(N), bv.reshape(N), cmb2, xp, zH, z1)
    out = _tc_d(H2p[0], H2p[1], s2p[0].reshape(N, 1), s2p[1].reshape(N, 1),
                gat_bias, x2, Wi1.T, Wh1.T, bi1, bh1, W_lin2.T, b_lin2,
                batch_f)
    return out
